# Initial kernel scaffold; baseline (speedup 1.0000x reference)
import jax
import jax.numpy as jnp
from jax.experimental import pallas as pl


def kernel(edge_index, emb, W1, b1, W2, b2):
    raise NotImplementedError("write your pallas kernel here")


# speedup vs baseline: 6.0679x; 6.0679x over previous
import functools

import jax
import jax.numpy as jnp
from jax import lax
from jax.experimental import pallas as pl
from jax.experimental.pallas import tpu as pltpu
from jax.experimental.pallas import tpu_sc as plsc

N_PAD = 10240
LANES = 16
CHUNK = 128
NTILES = 32
ROWS_PER_TILE = N_PAD // 16


def _mesh():
    return plsc.VectorSubcoreMesh(core_axis_name="c", subcore_axis_name="s",
                                  num_cores=2, num_subcores=16)


def _sc_degree(dst2d, zeros_ones):
    g_per_tile = dst2d.shape[0] // NTILES
    w = zeros_ones.shape[2]

    @functools.partial(
        pl.kernel,
        out_type=jax.ShapeDtypeStruct((2, N_PAD, w), jnp.float32),
        mesh=_mesh(),
        scratch_types=[
            pltpu.VMEM((g_per_tile, CHUNK), jnp.int32),
            pltpu.VMEM((2, CHUNK, w), jnp.float32),
            pltpu.VMEM_SHARED((N_PAD, w), jnp.float32),
        ],
    )
    def k(dst_hbm, zo_hbm, out_hbm, idx_v, zo_v, deg_sh):
        c = lax.axis_index("c")
        s = lax.axis_index("s")
        wid = c * 16 + s

        pltpu.sync_copy(zo_hbm, zo_v)
        for kk in range(ROWS_PER_TILE // CHUNK):
            pltpu.sync_copy(zo_v.at[0],
                            deg_sh.at[pl.ds(s * ROWS_PER_TILE + kk * CHUNK, CHUNK)])
        plsc.subcore_barrier()

        pltpu.sync_copy(dst_hbm.at[pl.ds(wid * g_per_tile, g_per_tile)], idx_v)

        def step(g, _):
            pltpu.sync_copy(zo_v.at[1], deg_sh.at[idx_v.at[g]], add=True)
            return 0
        lax.fori_loop(0, g_per_tile, step, 0)

        plsc.subcore_barrier()
        pltpu.sync_copy(deg_sh.at[pl.ds(s * ROWS_PER_TILE, ROWS_PER_TILE)],
                        out_hbm.at[c, pl.ds(s * ROWS_PER_TILE, ROWS_PER_TILE)])

    return k(dst2d, zeros_ones)


IDX_BLK = 16


def _sc_edge_pass(y, src2d, dst2d):
    g_per_tile = src2d.shape[0] // NTILES
    nblk = g_per_tile // IDX_BLK
    d = y.shape[1]

    @functools.partial(
        pl.kernel,
        out_type=jax.ShapeDtypeStruct((2, N_PAD, d), jnp.float32),
        mesh=_mesh(),
        scratch_types=[
            pltpu.VMEM((IDX_BLK, CHUNK), jnp.int32),
            pltpu.VMEM((IDX_BLK, CHUNK), jnp.int32),
            pltpu.VMEM((2, CHUNK, d), jnp.float32),
            pltpu.VMEM_SHARED((N_PAD, d), jnp.float32),
            pltpu.SemaphoreType.DMA,
            pltpu.SemaphoreType.DMA,
        ],
    )
    def k(y_hbm, src_hbm, dst_hbm, out_hbm, src_v, dst_v, rows_v, agg_sh, sem0, sem1):
        c = lax.axis_index("c")
        s = lax.axis_index("s")
        wid = c * 16 + s
        sems = [sem0, sem1]

        def zbody(i, _):
            r = i // (d // LANES)
            col = (i % (d // LANES)) * LANES
            rows_v[0, r, pl.ds(col, LANES)] = jnp.zeros((LANES,), jnp.float32)
            return 0
        lax.fori_loop(0, CHUNK * (d // LANES), zbody, 0)
        for kk in range(ROWS_PER_TILE // CHUNK):
            pltpu.sync_copy(rows_v.at[0], agg_sh.at[pl.ds(s * ROWS_PER_TILE + kk * CHUNK, CHUNK)])
        plsc.subcore_barrier()

        def blk_body(blk, _):
            base = wid * g_per_tile + blk * IDX_BLK
            pltpu.sync_copy(src_hbm.at[pl.ds(base, IDX_BLK)], src_v)
            pltpu.sync_copy(dst_hbm.at[pl.ds(base, IDX_BLK)], dst_v)

            pltpu.async_copy(y_hbm.at[src_v.at[0]], rows_v.at[0], sem0)

            def step2(t, _):
                for b in range(2):
                    g = t * 2 + b
                    pltpu.make_async_copy(y_hbm.at[src_v.at[g]], rows_v.at[b], sems[b]).wait()

                    @pl.when(g + 1 < IDX_BLK)
                    def _():
                        pltpu.async_copy(y_hbm.at[src_v.at[g + 1]], rows_v.at[1 - b],
                                         sems[1 - b])

                    pltpu.sync_copy(rows_v.at[b], agg_sh.at[dst_v.at[g]], add=True)
                return 0
            lax.fori_loop(0, IDX_BLK // 2, step2, 0)
            return 0
        lax.fori_loop(0, nblk, blk_body, 0)

        plsc.subcore_barrier()
        pltpu.sync_copy(agg_sh.at[pl.ds(s * ROWS_PER_TILE, ROWS_PER_TILE)],
                        out_hbm.at[c, pl.ds(s * ROWS_PER_TILE, ROWS_PER_TILE)])

    return k(y, src2d, dst2d)


def _dinv_block(deg_ref):
    deg = deg_ref[0, :, 0:1] + deg_ref[1, :, 0:1]
    return jnp.where(deg > 0, lax.rsqrt(deg), 0.0)


def _tc_in(x, deg_p, W):
    n, d = x.shape

    def body(x_ref, deg_ref, w_ref, y_ref):
        dinv = _dinv_block(deg_ref)
        y_ref[...] = dinv * jnp.dot(x_ref[...], w_ref[...],
                                    preferred_element_type=jnp.float32)

    return pl.pallas_call(
        body,
        grid=(n // 128,),
        in_specs=[
            pl.BlockSpec((128, d), lambda i: (i, 0)),
            pl.BlockSpec((2, 128, 128), lambda i: (0, i, 0)),
            pl.BlockSpec((d, d), lambda i: (0, 0)),
        ],
        out_specs=pl.BlockSpec((128, d), lambda i: (i, 0)),
        out_shape=jax.ShapeDtypeStruct((n, d), jnp.float32),
    )(x, deg_p, W)


def _tc_mid(agg_p, deg_p, b, W):
    n, d = agg_p.shape[1], agg_p.shape[2]

    def body(a_ref, deg_ref, b_ref, w_ref, y_ref):
        dinv = _dinv_block(deg_ref)
        x2 = jnp.maximum(dinv * (a_ref[0] + a_ref[1]) + b_ref[...], 0.0)
        y_ref[...] = dinv * jnp.dot(x2, w_ref[...], preferred_element_type=jnp.float32)

    return pl.pallas_call(
        body,
        grid=(n // 128,),
        in_specs=[
            pl.BlockSpec((2, 128, d), lambda i: (0, i, 0)),
            pl.BlockSpec((2, 128, 128), lambda i: (0, i, 0)),
            pl.BlockSpec((1, d), lambda i: (0, 0)),
            pl.BlockSpec((d, d), lambda i: (0, 0)),
        ],
        out_specs=pl.BlockSpec((128, d), lambda i: (i, 0)),
        out_shape=jax.ShapeDtypeStruct((n, d), jnp.float32),
    )(agg_p, deg_p, b, W)


def _tc_out(agg_p, deg_p, b):
    n, d = agg_p.shape[1], agg_p.shape[2]

    def body(a_ref, deg_ref, b_ref, o_ref):
        dinv = _dinv_block(deg_ref)
        o_ref[...] = jnp.maximum(dinv * (a_ref[0] + a_ref[1]) + b_ref[...], 0.0)

    return pl.pallas_call(
        body,
        grid=(n // 128,),
        in_specs=[
            pl.BlockSpec((2, 128, d), lambda i: (0, i, 0)),
            pl.BlockSpec((2, 128, 128), lambda i: (0, i, 0)),
            pl.BlockSpec((1, d), lambda i: (0, 0)),
        ],
        out_specs=pl.BlockSpec((128, d), lambda i: (i, 0)),
        out_shape=jax.ShapeDtypeStruct((n, d), jnp.float32),
    )(agg_p, deg_p, b)


def kernel(edge_index, emb, W1, b1, W2, b2):
    src, dst = edge_index[0], edge_index[1]
    e = src.shape[0]
    n, d = emb.shape

    unit = NTILES * CHUNK * IDX_BLK
    e_pad = ((e + unit - 1) // unit) * unit
    pad_idx = jnp.full((e_pad - e,), N_PAD - 1, jnp.int32)
    src_p = jnp.concatenate([src, pad_idx]).reshape(-1, CHUNK)
    dst_p = jnp.concatenate([dst, pad_idx]).reshape(-1, CHUNK)
    emb_pad = jnp.pad(emb, ((0, N_PAD - n), (0, 0)))
    b1r = b1.reshape(1, d)
    b2r = b2.reshape(1, d)

    zeros_ones = jnp.stack([jnp.zeros((CHUNK, d), jnp.float32),
                            jnp.ones((CHUNK, d), jnp.float32)])

    deg_p = _sc_degree(dst_p, zeros_ones)
    y1 = _tc_in(emb_pad, deg_p, W1)
    agg1 = _sc_edge_pass(y1, src_p, dst_p)
    y2 = _tc_mid(agg1, deg_p, b1r, W2)
    agg2 = _sc_edge_pass(y2, src_p, dst_p)
    out = _tc_out(agg2, deg_p, b2r)
    return out[:n]


# speedup vs baseline: 6.1199x; 1.0086x over previous
import functools

import jax
import jax.numpy as jnp
from jax import lax
from jax.experimental import pallas as pl
from jax.experimental.pallas import tpu as pltpu
from jax.experimental.pallas import tpu_sc as plsc

N_PAD = 10112
LANES = 16
CHUNK = 128
NTILES = 32
NBLOCKS = N_PAD // CHUNK
NBUF = 3


def _mesh():
    return plsc.VectorSubcoreMesh(core_axis_name="c", subcore_axis_name="s",
                                  num_cores=2, num_subcores=16)


def _tile_blocks(s):
    return [(k, s + k * 16) for k in range((NBLOCKS + 15) // 16)]


def _zero_vmem_block(buf_ref, d):
    def zbody(i, _):
        r = i // (d // LANES)
        col = (i % (d // LANES)) * LANES
        buf_ref[r, pl.ds(col, LANES)] = jnp.zeros((LANES,), jnp.float32)
        return 0
    lax.fori_loop(0, CHUNK * (d // LANES), zbody, 0)


def _sc_degree(dst1d, ones):
    g_per_tile = dst1d.shape[0] // (NTILES * CHUNK)
    d = ones.shape[1]

    @functools.partial(
        pl.kernel,
        out_type=jax.ShapeDtypeStruct((2, N_PAD, d), jnp.float32),
        mesh=_mesh(),
        scratch_types=[
            pltpu.VMEM((NBUF, CHUNK), jnp.int32),
            pltpu.VMEM((CHUNK, d), jnp.float32),
            pltpu.VMEM_SHARED((N_PAD, d), jnp.float32),
            [pltpu.SemaphoreType.DMA] * NBUF,
        ],
    )
    def k(dst_hbm, ones_hbm, out_hbm, ib_v, ones_v, deg_sh, isems):
        c = lax.axis_index("c")
        s = lax.axis_index("s")
        wid = c * 16 + s
        base = wid * g_per_tile

        _zero_vmem_block(ones_v, d)
        for _, blk in _tile_blocks(s):
            @pl.when(blk < NBLOCKS)
            def _():
                pltpu.sync_copy(ones_v, deg_sh.at[pl.ds(blk * CHUNK, CHUNK)])
        plsc.subcore_barrier()
        pltpu.sync_copy(ones_hbm, ones_v)

        def idx_load(g, b):
            pltpu.async_copy(dst_hbm.at[pl.ds((base + g) * CHUNK, CHUNK)],
                             ib_v.at[b], isems[b])

        for j in range(NBUF):
            idx_load(j, j)

        def step(t, _):
            for b in range(NBUF):
                g = t * NBUF + b
                pltpu.make_async_copy(dst_hbm.at[pl.ds((base + g) * CHUNK, CHUNK)],
                                      ib_v.at[b], isems[b]).wait()
                pltpu.sync_copy(ones_v, deg_sh.at[ib_v.at[b]], add=True)

                @pl.when(g + NBUF < g_per_tile)
                def _():
                    idx_load(g + NBUF, b)
            return 0
        lax.fori_loop(0, g_per_tile // NBUF, step, 0)

        plsc.subcore_barrier()
        for _, blk in _tile_blocks(s):
            @pl.when(blk < NBLOCKS)
            def _():
                pltpu.sync_copy(deg_sh.at[pl.ds(blk * CHUNK, CHUNK)],
                                out_hbm.at[c, pl.ds(blk * CHUNK, CHUNK)])

    return k(dst1d, ones)


def _sc_edge_pass(y, src1d, dst1d):
    g_per_tile = src1d.shape[0] // (NTILES * CHUNK)
    assert g_per_tile % NBUF == 0
    d = y.shape[1]

    @functools.partial(
        pl.kernel,
        out_type=jax.ShapeDtypeStruct((2, N_PAD, d), jnp.float32),
        mesh=_mesh(),
        scratch_types=[
            pltpu.VMEM((NBUF, 2, CHUNK), jnp.int32),
            pltpu.VMEM((NBUF, CHUNK, d), jnp.float32),
            pltpu.VMEM_SHARED((N_PAD, d), jnp.float32),
            [pltpu.SemaphoreType.DMA] * NBUF,
            [pltpu.SemaphoreType.DMA] * NBUF,
        ],
    )
    def k(y_hbm, src_hbm, dst_hbm, out_hbm, ib_v, rows_v, agg_sh, gsems, isems):
        c = lax.axis_index("c")
        s = lax.axis_index("s")
        wid = c * 16 + s
        base = wid * g_per_tile

        _zero_vmem_block(rows_v.at[0], d)
        for _, blk in _tile_blocks(s):
            @pl.when(blk < NBLOCKS)
            def _():
                pltpu.sync_copy(rows_v.at[0], agg_sh.at[pl.ds(blk * CHUNK, CHUNK)])
        plsc.subcore_barrier()

        def idx_load(g, b):
            off = (base + g) * CHUNK
            pltpu.async_copy(src_hbm.at[pl.ds(off, CHUNK)], ib_v.at[b, 0], isems[b])
            pltpu.async_copy(dst_hbm.at[pl.ds(off, CHUNK)], ib_v.at[b, 1], isems[b])

        def idx_wait(g, b):
            off = (base + g) * CHUNK
            pltpu.make_async_copy(src_hbm.at[pl.ds(off, CHUNK)], ib_v.at[b, 0],
                                  isems[b]).wait()
            pltpu.make_async_copy(dst_hbm.at[pl.ds(off, CHUNK)], ib_v.at[b, 1],
                                  isems[b]).wait()

        def gather(g, b):
            pltpu.async_copy(y_hbm.at[ib_v.at[b, 0]], rows_v.at[b], gsems[b])

        for j in range(NBUF):
            idx_load(j, j)
        for j in range(NBUF - 1):
            idx_wait(j, j)
            gather(j, j)

        def step(t, _):
            for j in range(NBUF):
                g = t * NBUF + j
                b = j
                pltpu.make_async_copy(y_hbm.at[ib_v.at[b, 0]], rows_v.at[b],
                                      gsems[b]).wait()
                pltpu.sync_copy(rows_v.at[b], agg_sh.at[ib_v.at[b, 1]],
                                add=True)

                @pl.when(g + NBUF < g_per_tile)
                def _():
                    idx_load(g + NBUF, b)

                bn = (j + NBUF - 1) % NBUF
                @pl.when(g + NBUF - 1 < g_per_tile)
                def _():
                    idx_wait(g + NBUF - 1, bn)
                    gather(g + NBUF - 1, bn)
            return 0
        lax.fori_loop(0, g_per_tile // NBUF, step, 0)

        plsc.subcore_barrier()
        for _, blk in _tile_blocks(s):
            @pl.when(blk < NBLOCKS)
            def _():
                pltpu.sync_copy(agg_sh.at[pl.ds(blk * CHUNK, CHUNK)],
                                out_hbm.at[c, pl.ds(blk * CHUNK, CHUNK)])

    return k(y, src1d, dst1d)


def _dinv_block(deg_ref):
    deg = deg_ref[0, :, 0:1] + deg_ref[1, :, 0:1]
    return jnp.where(deg > 0, lax.rsqrt(deg), 0.0)


def _tc_in(x, deg_p, W):
    n, d = x.shape

    def body(x_ref, deg_ref, w_ref, y_ref):
        dinv = _dinv_block(deg_ref)
        y_ref[...] = dinv * jnp.dot(x_ref[...], w_ref[...],
                                    preferred_element_type=jnp.float32)

    return pl.pallas_call(
        body,
        grid=(n // 128,),
        in_specs=[
            pl.BlockSpec((128, d), lambda i: (i, 0)),
            pl.BlockSpec((2, 128, 128), lambda i: (0, i, 0)),
            pl.BlockSpec((d, d), lambda i: (0, 0)),
        ],
        out_specs=pl.BlockSpec((128, d), lambda i: (i, 0)),
        out_shape=jax.ShapeDtypeStruct((n, d), jnp.float32),
    )(x, deg_p, W)


def _tc_mid(agg_p, deg_p, b, W):
    n, d = agg_p.shape[1], agg_p.shape[2]

    def body(a_ref, deg_ref, b_ref, w_ref, y_ref):
        dinv = _dinv_block(deg_ref)
        x2 = jnp.maximum(dinv * (a_ref[0] + a_ref[1]) + b_ref[...], 0.0)
        y_ref[...] = dinv * jnp.dot(x2, w_ref[...], preferred_element_type=jnp.float32)

    return pl.pallas_call(
        body,
        grid=(n // 128,),
        in_specs=[
            pl.BlockSpec((2, 128, d), lambda i: (0, i, 0)),
            pl.BlockSpec((2, 128, 128), lambda i: (0, i, 0)),
            pl.BlockSpec((1, d), lambda i: (0, 0)),
            pl.BlockSpec((d, d), lambda i: (0, 0)),
        ],
        out_specs=pl.BlockSpec((128, d), lambda i: (i, 0)),
        out_shape=jax.ShapeDtypeStruct((n, d), jnp.float32),
    )(agg_p, deg_p, b, W)


def _tc_out(agg_p, deg_p, b):
    n, d = agg_p.shape[1], agg_p.shape[2]

    def body(a_ref, deg_ref, b_ref, o_ref):
        dinv = _dinv_block(deg_ref)
        o_ref[...] = jnp.maximum(dinv * (a_ref[0] + a_ref[1]) + b_ref[...], 0.0)

    return pl.pallas_call(
        body,
        grid=(n // 128,),
        in_specs=[
            pl.BlockSpec((2, 128, d), lambda i: (0, i, 0)),
            pl.BlockSpec((2, 128, 128), lambda i: (0, i, 0)),
            pl.BlockSpec((1, d), lambda i: (0, 0)),
        ],
        out_specs=pl.BlockSpec((128, d), lambda i: (i, 0)),
        out_shape=jax.ShapeDtypeStruct((n, d), jnp.float32),
    )(agg_p, deg_p, b)


def kernel(edge_index, emb, W1, b1, W2, b2):
    src, dst = edge_index[0], edge_index[1]
    e = src.shape[0]
    n, d = emb.shape

    unit = NTILES * CHUNK * NBUF
    e_pad = ((e + unit - 1) // unit) * unit
    src_p = jnp.concatenate([src, jnp.full((e_pad - e,), n, jnp.int32)])
    dst_p = jnp.concatenate([dst, jnp.full((e_pad - e,), N_PAD - 1, jnp.int32)])
    emb_pad = jnp.pad(emb, ((0, N_PAD - n), (0, 0)))
    b1r = b1.reshape(1, d)
    b2r = b2.reshape(1, d)
    ones = jnp.ones((CHUNK, d), jnp.float32)

    deg_p = _sc_degree(dst_p, ones)
    y1 = _tc_in(emb_pad, deg_p, W1)
    agg1 = _sc_edge_pass(y1, src_p, dst_p)
    y2 = _tc_mid(agg1, deg_p, b1r, W2)
    agg2 = _sc_edge_pass(y2, src_p, dst_p)
    out = _tc_out(agg2, deg_p, b2r)
    return out[:n]
